# baseline (speedup 1.0000x reference)
import functools

import jax
import jax.numpy as jnp
from jax.experimental import pallas as pl
from jax.experimental.pallas import tpu as pltpu


def _moe_fused_kernel(x_ref, gw_ref, gb_ref, w1_ref, b1_ref, w2cat_ref,
                      b2_ref, out_ref, h_ref, *, num_experts, hidden):
    x = x_ref[...]
    tb = x.shape[0]
    n_exp = num_experts

    lane = jax.lax.broadcasted_iota(jnp.int32, (tb, n_exp), 1)
    logits = (jnp.dot(x, gw_ref[...], preferred_element_type=jnp.float32)
              + gb_ref[...])

    m1 = jnp.max(logits, axis=-1, keepdims=True)
    i1 = jnp.min(jnp.where(logits == m1, lane, n_exp), axis=-1, keepdims=True)
    rest = jnp.where(lane == i1, -jnp.inf, logits)
    m2 = jnp.max(rest, axis=-1, keepdims=True)
    i2 = jnp.min(jnp.where(rest == m2, lane, n_exp), axis=-1, keepdims=True)

    e2 = jnp.exp(m2 - m1)
    c1 = 1.0 / (1.0 + e2)
    c2 = 1.0 - c1

    for e in range(n_exp):
        he = (jnp.dot(x, w1_ref[e], preferred_element_type=jnp.float32)
              + b1_ref[e])
        he = jnp.maximum(he, 0.0)
        ce = jnp.where(i1 == e, c1, 0.0) + jnp.where(i2 == e, c2, 0.0)
        h_ref[:, e * hidden:(e + 1) * hidden] = he * ce

    cmat = jnp.where(lane == i1, c1, 0.0) + jnp.where(lane == i2, c2, 0.0)
    y = jnp.dot(h_ref[...], w2cat_ref[...], preferred_element_type=jnp.float32)
    y = y + jnp.dot(cmat, b2_ref[...], preferred_element_type=jnp.float32)
    out_ref[...] = y


def kernel(x, gate_w, gate_b, w1, b1, w2, b2):
    batch, d_in = x.shape
    num_experts, _, hidden = w1.shape
    d_out = w2.shape[2]

    if batch >= 1024:
        batch_tile = 512
    else:
        batch_tile = max(8, ((batch + 7) // 8) * 8)
    n_tiles = pl.cdiv(batch, batch_tile)
    padded = n_tiles * batch_tile
    if padded != batch:
        x = jnp.pad(x, ((0, padded - batch), (0, 0)))

    x_c = x.astype(jnp.float32)
    gw = gate_w.astype(jnp.float32)
    gb = gate_b.reshape(1, num_experts).astype(jnp.float32)
    w1_c = w1.astype(jnp.float32)
    b1_3 = b1.reshape(num_experts, 1, hidden).astype(jnp.float32)
    w2cat = w2.astype(jnp.float32).reshape(num_experts * hidden, d_out)
    b2_2 = b2.astype(jnp.float32)

    body = functools.partial(_moe_fused_kernel, num_experts=num_experts,
                             hidden=hidden)

    flops = 2 * padded * (d_in * num_experts
                          + num_experts * (d_in * hidden + hidden * d_out))
    bytes_accessed = 4 * (padded * (d_in + d_out)
                          + num_experts * (d_in * hidden + hidden * d_out)
                          + d_in * num_experts
                          + num_experts * (1 + hidden + d_out))
    cost = pl.CostEstimate(flops=int(flops),
                           transcendentals=int(padded),
                           bytes_accessed=int(bytes_accessed))

    out = pl.pallas_call(
        body,
        out_shape=jax.ShapeDtypeStruct((padded, d_out), jnp.float32),
        grid=(n_tiles,),
        in_specs=[
            pl.BlockSpec((batch_tile, d_in), lambda i: (i, 0)),
            pl.BlockSpec((d_in, num_experts), lambda i: (0, 0)),
            pl.BlockSpec((1, num_experts), lambda i: (0, 0)),
            pl.BlockSpec((num_experts, d_in, hidden), lambda i: (0, 0, 0)),
            pl.BlockSpec((num_experts, 1, hidden), lambda i: (0, 0, 0)),
            pl.BlockSpec((num_experts * hidden, d_out), lambda i: (0, 0)),
            pl.BlockSpec((num_experts, d_out), lambda i: (0, 0)),
        ],
        out_specs=pl.BlockSpec((batch_tile, d_out), lambda i: (i, 0)),
        scratch_shapes=[
            pltpu.VMEM((batch_tile, num_experts * hidden), jnp.float32),
        ],
        compiler_params=pltpu.CompilerParams(
            dimension_semantics=("parallel",),
            vmem_limit_bytes=60 * 1024 * 1024),
        cost_estimate=cost,
    )(x_c, gw, gb, w1_c, b1_3, w2cat, b2_2)
    return out[:batch]


# speedup vs baseline: 1.3366x; 1.3366x over previous
import functools

import jax
import jax.numpy as jnp
from jax.experimental import pallas as pl
from jax.experimental.pallas import tpu as pltpu


def _moe_fused_kernel(x_ref, gw_ref, gb_ref, w1_ref, b1_ref, w2cat_ref,
                      b2_ref, out_ref, h_ref, *, num_experts, hidden):
    x = x_ref[...]
    tb = x.shape[0]
    n_exp = num_experts

    lane = jax.lax.broadcasted_iota(jnp.int32, (tb, n_exp), 1)
    logits = (jnp.dot(x, gw_ref[...], preferred_element_type=jnp.float32)
              + gb_ref[...])

    m1 = jnp.max(logits, axis=-1, keepdims=True)
    i1 = jnp.min(jnp.where(logits == m1, lane, n_exp), axis=-1, keepdims=True)
    rest = jnp.where(lane == i1, -jnp.inf, logits)
    m2 = jnp.max(rest, axis=-1, keepdims=True)
    i2 = jnp.min(jnp.where(rest == m2, lane, n_exp), axis=-1, keepdims=True)

    e2 = jnp.exp(m2 - m1)
    c1 = 1.0 / (1.0 + e2)
    c2 = 1.0 - c1

    xb = x.astype(jnp.bfloat16)

    for e in range(n_exp):
        he = (jnp.dot(xb, w1_ref[e], preferred_element_type=jnp.float32)
              + b1_ref[e])
        he = jnp.maximum(he, 0.0)
        ce = jnp.where(i1 == e, c1, 0.0) + jnp.where(i2 == e, c2, 0.0)
        h_ref[:, e * hidden:(e + 1) * hidden] = (he * ce).astype(jnp.bfloat16)

    cmat = jnp.where(lane == i1, c1, 0.0) + jnp.where(lane == i2, c2, 0.0)
    y = jnp.dot(h_ref[...], w2cat_ref[...], preferred_element_type=jnp.float32)
    y = y + jnp.dot(cmat, b2_ref[...], preferred_element_type=jnp.float32)
    out_ref[...] = y


def kernel(x, gate_w, gate_b, w1, b1, w2, b2):
    batch, d_in = x.shape
    num_experts, _, hidden = w1.shape
    d_out = w2.shape[2]

    if batch >= 2048:
        batch_tile = 1024
    else:
        batch_tile = max(8, ((batch + 7) // 8) * 8)
    n_tiles = pl.cdiv(batch, batch_tile)
    padded = n_tiles * batch_tile
    if padded != batch:
        x = jnp.pad(x, ((0, padded - batch), (0, 0)))

    x_c = x.astype(jnp.float32)
    gw = gate_w.astype(jnp.float32)
    gb = gate_b.reshape(1, num_experts).astype(jnp.float32)
    w1_c = w1.astype(jnp.bfloat16)
    b1_3 = b1.reshape(num_experts, 1, hidden).astype(jnp.float32)
    w2cat = w2.astype(jnp.bfloat16).reshape(num_experts * hidden, d_out)
    b2_2 = b2.astype(jnp.float32)

    body = functools.partial(_moe_fused_kernel, num_experts=num_experts,
                             hidden=hidden)

    flops = 2 * padded * (d_in * num_experts
                          + num_experts * (d_in * hidden + hidden * d_out))
    bytes_accessed = 4 * (padded * (d_in + d_out)
                          + num_experts * (d_in * hidden + hidden * d_out)
                          + d_in * num_experts
                          + num_experts * (1 + hidden + d_out))
    cost = pl.CostEstimate(flops=int(flops),
                           transcendentals=int(padded),
                           bytes_accessed=int(bytes_accessed))

    out = pl.pallas_call(
        body,
        out_shape=jax.ShapeDtypeStruct((padded, d_out), jnp.float32),
        grid=(n_tiles,),
        in_specs=[
            pl.BlockSpec((batch_tile, d_in), lambda i: (i, 0)),
            pl.BlockSpec((d_in, num_experts), lambda i: (0, 0)),
            pl.BlockSpec((1, num_experts), lambda i: (0, 0)),
            pl.BlockSpec((num_experts, d_in, hidden), lambda i: (0, 0, 0)),
            pl.BlockSpec((num_experts, 1, hidden), lambda i: (0, 0, 0)),
            pl.BlockSpec((num_experts * hidden, d_out), lambda i: (0, 0)),
            pl.BlockSpec((num_experts, d_out), lambda i: (0, 0)),
        ],
        out_specs=pl.BlockSpec((batch_tile, d_out), lambda i: (i, 0)),
        scratch_shapes=[
            pltpu.VMEM((batch_tile, num_experts * hidden), jnp.bfloat16),
        ],
        compiler_params=pltpu.CompilerParams(
            dimension_semantics=("parallel",),
            vmem_limit_bytes=60 * 1024 * 1024),
        cost_estimate=cost,
    )(x_c, gw, gb, w1_c, b1_3, w2cat, b2_2)
    return out[:batch]
